# baseline (speedup 1.0000x reference)
import functools

import jax
import jax.numpy as jnp
from jax import lax
from jax.experimental import pallas as pl
from jax.experimental.pallas import tpu as pltpu
from jax.experimental.pallas import tpu_sc as plsc

_LANES = 16
_CHUNK = 128


@functools.partial(jax.jit, static_argnames=("batch", "dim"))
def _run(user_t, item_t, users2d, items2d, *, batch, dim):
    info = plsc.get_sparse_core_info()
    n_workers = info.num_cores * info.num_subcores
    b_per_w = batch // n_workers
    n_chunks = b_per_w // _CHUNK
    n_vecs = b_per_w // _LANES

    mesh = plsc.VectorSubcoreMesh(core_axis_name="c", subcore_axis_name="s")

    @functools.partial(
        pl.kernel,
        out_type=jax.ShapeDtypeStruct((batch,), jnp.float32),
        mesh=mesh,
        scratch_types=[
            pltpu.VMEM((n_chunks, _CHUNK), jnp.int32),
            pltpu.VMEM((n_chunks, _CHUNK), jnp.int32),
            pltpu.VMEM((dim, b_per_w), jnp.float32),
            pltpu.VMEM((dim, b_per_w), jnp.float32),
            pltpu.VMEM((b_per_w,), jnp.float32),
            pltpu.SemaphoreType.DMA,
        ],
        compiler_params=pltpu.CompilerParams(
            needs_layout_passes=False, use_tc_tiling_on_sc=False),
    )
    def sc_kernel(ut_hbm, it_hbm, users_hbm, items_hbm, out_hbm,
                  uidx_v, iidx_v, urows_v, irows_v, scores_v, sem):
        wid = lax.axis_index("s") * info.num_cores + lax.axis_index("c")
        idx_row0 = wid * n_chunks

        pltpu.sync_copy(users_hbm.at[pl.ds(idx_row0, n_chunks)], uidx_v)
        pltpu.sync_copy(items_hbm.at[pl.ds(idx_row0, n_chunks)], iidx_v)

        copies = []
        for d in range(dim):
            for j in range(n_chunks):
                dst = pl.ds(j * _CHUNK, _CHUNK)
                copies.append(pltpu.async_copy(
                    ut_hbm.at[d].at[uidx_v.at[j]], urows_v.at[d, dst], sem))
                copies.append(pltpu.async_copy(
                    it_hbm.at[d].at[iidx_v.at[j]], irows_v.at[d, dst], sem))
        for c in copies:
            c.wait()

        def chunk_body(c, carry):
            base = pl.multiple_of(c * _LANES, _LANES)
            sl = pl.ds(base, _LANES)
            acc = jnp.zeros((_LANES,), jnp.float32)
            for d in range(dim):
                acc = acc + urows_v[d, sl] * irows_v[d, sl]
            scores_v[sl] = acc
            return carry

        lax.fori_loop(0, n_vecs, chunk_body, 0)

        pltpu.sync_copy(scores_v, out_hbm.at[pl.ds(wid * b_per_w, b_per_w)])

    return sc_kernel(user_t, item_t, users2d, items2d)


def kernel(user_table, item_table, users, items):
    batch = users.shape[0]
    dim = user_table.shape[1]
    users2d = users.astype(jnp.int32).reshape(batch // _CHUNK, _CHUNK)
    items2d = items.astype(jnp.int32).reshape(batch // _CHUNK, _CHUNK)
    return _run(user_table.T, item_table.T, users2d, items2d,
                batch=batch, dim=dim)


# speedup vs baseline: 1.0083x; 1.0083x over previous
import functools

import jax
import jax.numpy as jnp
from jax import lax
from jax.experimental import pallas as pl
from jax.experimental.pallas import tpu as pltpu
from jax.experimental.pallas import tpu_sc as plsc

_LANES = 16
_CHUNK = 128


@functools.partial(jax.jit, static_argnames=("batch", "dim", "n_rows"))
def _run(user_f, item_f, users2d, items2d, *, batch, dim, n_rows):
    info = plsc.get_sparse_core_info()
    n_workers = info.num_cores * info.num_subcores
    b_per_w = batch // n_workers
    n_chunks = b_per_w // _CHUNK
    n_vecs = _CHUNK // _LANES
    flat_rows = dim * n_chunks

    mesh = plsc.VectorSubcoreMesh(core_axis_name="c", subcore_axis_name="s")

    @functools.partial(
        pl.kernel,
        out_type=jax.ShapeDtypeStruct((batch,), jnp.float32),
        mesh=mesh,
        scratch_types=[
            pltpu.VMEM((n_chunks, _CHUNK), jnp.int32),
            pltpu.VMEM((n_chunks, _CHUNK), jnp.int32),
            pltpu.VMEM((flat_rows * _CHUNK,), jnp.int32),
            pltpu.VMEM((flat_rows * _CHUNK,), jnp.int32),
            pltpu.VMEM((flat_rows * _CHUNK,), jnp.float32),
            pltpu.VMEM((flat_rows * _CHUNK,), jnp.float32),
            pltpu.VMEM((b_per_w,), jnp.float32),
            pltpu.SemaphoreType.DMA,
        ],
        compiler_params=pltpu.CompilerParams(
            needs_layout_passes=False, use_tc_tiling_on_sc=False),
    )
    def sc_kernel(ut_hbm, it_hbm, users_hbm, items_hbm, out_hbm,
                  uidx_v, iidx_v, uflat_v, iflat_v, urows_v, irows_v,
                  scores_v, sem):
        wid = lax.axis_index("s") * info.num_cores + lax.axis_index("c")
        idx_row0 = wid * n_chunks

        pltpu.sync_copy(users_hbm.at[pl.ds(idx_row0, n_chunks)], uidx_v)
        pltpu.sync_copy(items_hbm.at[pl.ds(idx_row0, n_chunks)], iidx_v)

        for d in range(dim):
            off = jnp.full((_LANES,), d * n_rows, jnp.int32)
            for j in range(n_chunks):
                for v in range(n_vecs):
                    sl = pl.ds(v * _LANES, _LANES)
                    dst = pl.ds((d * n_chunks + j) * _CHUNK + v * _LANES,
                                _LANES)
                    uflat_v[dst] = uidx_v[j, sl] + off
                    iflat_v[dst] = iidx_v[j, sl] + off

        cu = pltpu.async_copy(ut_hbm.at[uflat_v], urows_v, sem)
        ci = pltpu.async_copy(it_hbm.at[iflat_v], irows_v, sem)
        cu.wait()
        ci.wait()

        def chunk_body(c, carry):
            base = pl.multiple_of(c * _LANES, _LANES)
            acc = jnp.zeros((_LANES,), jnp.float32)
            for d in range(dim):
                sl = pl.ds(d * b_per_w + base, _LANES)
                acc = acc + urows_v[sl] * irows_v[sl]
            scores_v[pl.ds(base, _LANES)] = acc
            return carry

        lax.fori_loop(0, b_per_w // _LANES, chunk_body, 0, unroll=False)

        pltpu.sync_copy(scores_v, out_hbm.at[pl.ds(wid * b_per_w, b_per_w)])

    return sc_kernel(user_f, item_f, users2d, items2d)


def kernel(user_table, item_table, users, items):
    batch = users.shape[0]
    n_rows, dim = user_table.shape
    users2d = users.astype(jnp.int32).reshape(batch // _CHUNK, _CHUNK)
    items2d = items.astype(jnp.int32).reshape(batch // _CHUNK, _CHUNK)
    user_f = user_table.T.reshape(n_rows * dim)
    item_f = item_table.T.reshape(n_rows * dim)
    return _run(user_f, item_f, users2d, items2d,
                batch=batch, dim=dim, n_rows=n_rows)
